# baseline (speedup 1.0000x reference)
import jax
import jax.numpy as jnp
from jax import lax
from jax.experimental import pallas as pl

_H = 256
_W = 256
_K = 18
_BH = 64


def _box_kernel(rlo_ref, clo_ref, out_ref):
    rlo = rlo_ref[0]
    clo = clo_ref[0]
    base = pl.program_id(1) * _BH
    ri = base + lax.broadcasted_iota(jnp.int32, (_BH, 1, _K), 0)
    rd = (ri - rlo[None]).astype(jnp.uint32)
    rowf = jnp.where(rd <= 10, jnp.float32(1.0), jnp.float32(0.0))
    cj = lax.broadcasted_iota(jnp.int32, (1, _W, _K), 1)
    cd = (cj - clo[None]).astype(jnp.uint32)
    colf = jnp.where(cd <= 10, jnp.float32(1.0), jnp.float32(0.0))
    out_ref[0] = rowf * colf


def kernel(x):
    b, k, _ = x.shape
    rlo = (x[:, :, 0] - 5)[:, None, :]
    clo = (x[:, :, 1] - 5)[:, None, :]

    out = pl.pallas_call(
        _box_kernel,
        grid=(b, _H // _BH),
        in_specs=[
            pl.BlockSpec((1, 1, k), lambda bi, hi: (bi, 0, 0)),
            pl.BlockSpec((1, 1, k), lambda bi, hi: (bi, 0, 0)),
        ],
        out_specs=pl.BlockSpec((1, _BH, _W, k), lambda bi, hi: (bi, hi, 0, 0)),
        out_shape=jax.ShapeDtypeStruct((b, _H, _W, k), jnp.float32),
    )(rlo, clo)
    return out


# speedup vs baseline: 2.7811x; 2.7811x over previous
import jax
import jax.numpy as jnp
from jax import lax
from jax.experimental import pallas as pl

_H = 256
_W = 256
_K = 18
_BH = 256


def _box_kernel(rlo_ref, clo_ref, out_ref):
    wk = _W * _K
    lane = lax.broadcasted_iota(jnp.int32, (1, wk), 1)
    j_id = lane // _K
    cd = (j_id - clo_ref[0]).astype(jnp.uint32)
    colf = jnp.where(cd <= 10, jnp.float32(1.0), jnp.float32(0.0))
    base = pl.program_id(1) * _BH
    ri = base + lax.broadcasted_iota(jnp.int32, (_BH, wk), 0)
    rd = (ri - rlo_ref[0]).astype(jnp.uint32)
    out_ref[0] = jnp.where(rd <= 10, colf, jnp.float32(0.0))


def kernel(x):
    b, k, _ = x.shape
    wk = _W * _K
    rlo = jnp.broadcast_to((x[:, :, 0] - 5)[:, None, :], (b, _W, k)).reshape(b, 1, wk)
    clo = jnp.broadcast_to((x[:, :, 1] - 5)[:, None, :], (b, _W, k)).reshape(b, 1, wk)

    out = pl.pallas_call(
        _box_kernel,
        grid=(b, _H // _BH),
        in_specs=[
            pl.BlockSpec((1, 1, wk), lambda bi, hi: (bi, 0, 0)),
            pl.BlockSpec((1, 1, wk), lambda bi, hi: (bi, 0, 0)),
        ],
        out_specs=pl.BlockSpec((1, _BH, wk), lambda bi, hi: (bi, hi, 0)),
        out_shape=jax.ShapeDtypeStruct((b, _H, wk), jnp.float32),
    )(rlo, clo)
    return out.reshape(b, _H, _W, k)
